# Initial kernel scaffold; baseline (speedup 1.0000x reference)
import functools

import jax
import jax.numpy as jnp
from jax import lax
from jax.experimental import pallas as pl
from jax.experimental.pallas import tpu as pltpu
from jax.experimental.pallas import tpu_sc as plsc

NUM_HASHES = 4
CLUSTER = 128

SC_CORES = 2
SC_SUBCORES = 16
NW = SC_CORES * SC_SUBCORES

CH = 128


def _sc_mesh():
    return plsc.VectorSubcoreMesh(
        core_axis_name="c", subcore_axis_name="s",
        num_cores=SC_CORES, num_subcores=SC_SUBCORES)


def _wid():
    return lax.axis_index("s") * SC_CORES + lax.axis_index("c")


def _gather_sorted(qf, kf, vf, qidx, kidx):
    total, d = qf.shape[0] * NUM_HASHES, qf.shape[1]
    rows_per_w = total // NW

    def body(qf_h, kf_h, vf_h, qidx_h, kidx_h, sq_h, sk_h, sv_h,
             iq_v, ik_v, qr_v, kr_v, vr_v, sem):
        base0 = _wid() * rows_per_w

        def chunk(c, carry):
            base = base0 + c * CH
            pltpu.sync_copy(qidx_h.at[pl.ds(base, CH)], iq_v)
            pltpu.async_copy(qf_h.at[iq_v], qr_v, sem).wait()
            pltpu.sync_copy(qr_v, sq_h.at[pl.ds(base, CH)])
            pltpu.sync_copy(kidx_h.at[pl.ds(base, CH)], ik_v)
            pltpu.async_copy(kf_h.at[ik_v], kr_v, sem).wait()
            pltpu.sync_copy(kr_v, sk_h.at[pl.ds(base, CH)])
            pltpu.async_copy(vf_h.at[ik_v], vr_v, sem).wait()
            pltpu.sync_copy(vr_v, sv_h.at[pl.ds(base, CH)])
            return carry

        lax.fori_loop(0, rows_per_w // CH, chunk, 0)

    out_t = jax.ShapeDtypeStruct((total, d), jnp.float32)
    fn = pl.kernel(
        body,
        out_type=(out_t, out_t, out_t),
        mesh=_sc_mesh(),
        scratch_types=[
            pltpu.VMEM((CH,), jnp.int32),
            pltpu.VMEM((CH,), jnp.int32),
            pltpu.VMEM((CH, d), jnp.float32),
            pltpu.VMEM((CH, d), jnp.float32),
            pltpu.VMEM((CH, d), jnp.float32),
            pltpu.SemaphoreType.DMA,
        ],
    )
    return fn(qf, kf, vf, qidx, kidx)


def _attn_body(nc, sq_ref, sk_ref, sv_ref, so_ref, lse_ref):
    for j in range(nc):
        q = sq_ref[j]
        k = sk_ref[j]
        v = sv_ref[j]
        s = lax.dot_general(q, k, (((1,), (1,)), ((), ())),
                            preferred_element_type=jnp.float32)
        m = jnp.max(s, axis=1, keepdims=True)
        p = jnp.exp(s - m)
        den = jnp.sum(p, axis=1, keepdims=True)
        o = jnp.dot(p, v, preferred_element_type=jnp.float32) / den
        so_ref[j] = o
        lse_ref[j, :] = (m + jnp.log(den))[:, 0]


def _cluster_attention(sq, sk, sv):
    nclusters, c, d = sq.shape
    CB = 8
    grid = (nclusters // CB,)
    blk = pl.BlockSpec((CB, c, d), lambda i: (i, 0, 0))
    lse_blk = pl.BlockSpec((CB, c), lambda i: (i, 0))
    so, lse = pl.pallas_call(
        functools.partial(_attn_body, CB),
        grid=grid,
        in_specs=[blk, blk, blk],
        out_specs=[blk, lse_blk],
        out_shape=[
            jax.ShapeDtypeStruct((nclusters, c, d), jnp.float32),
            jax.ShapeDtypeStruct((nclusters, c), jnp.float32),
        ],
    )(sq, sk, sv)
    return so, lse


def _unsort(so_flat, lse_flat, qpos_flat, n):
    total, d = so_flat.shape
    segs = total // n
    segs_per_w = segs // NW

    def body(so_h, lse_h, qpos_h, oun_h, logits_h,
             qpos_v, lse_v, inv_v, gidx_v, log_v, rows_v, sem):
        w = _wid()

        def seg(t, carry):
            s = w * segs_per_w + t
            seg_base = s * n
            pltpu.sync_copy(qpos_h.at[pl.ds(seg_base, n)], qpos_v)
            pltpu.sync_copy(lse_h.at[pl.ds(seg_base, n)], lse_v)

            def inv_chunk(i, c2):
                idx = qpos_v[pl.ds(i * 16, 16)]
                vals = lax.iota(jnp.int32, 16) + i * 16
                plsc.store_scatter(inv_v, [idx], vals)
                return c2

            lax.fori_loop(0, n // 16, inv_chunk, 0)

            def g_chunk(i, c2):
                g = inv_v[pl.ds(i * 16, 16)]
                log_v[pl.ds(i * 16, 16)] = plsc.load_gather(lse_v, [g])
                gidx_v[pl.ds(i * 16, 16)] = g + seg_base
                return c2

            lax.fori_loop(0, n // 16, g_chunk, 0)
            pltpu.sync_copy(log_v, logits_h.at[pl.ds(seg_base, n)])

            def row_chunk(c, c2):
                base = seg_base + c * CH
                pltpu.async_copy(
                    so_h.at[gidx_v.at[pl.ds(c * CH, CH)]], rows_v, sem
                ).wait()
                pltpu.sync_copy(rows_v, oun_h.at[pl.ds(base, CH)])
                return c2

            lax.fori_loop(0, n // CH, row_chunk, 0)
            return carry

        lax.fori_loop(0, segs_per_w, seg, 0)

    fn = pl.kernel(
        body,
        out_type=(
            jax.ShapeDtypeStruct((total, d), jnp.float32),
            jax.ShapeDtypeStruct((total,), jnp.float32),
        ),
        mesh=_sc_mesh(),
        scratch_types=[
            pltpu.VMEM((n,), jnp.int32),
            pltpu.VMEM((n,), jnp.float32),
            pltpu.VMEM((n,), jnp.int32),
            pltpu.VMEM((n,), jnp.int32),
            pltpu.VMEM((n,), jnp.float32),
            pltpu.VMEM((CH, d), jnp.float32),
            pltpu.SemaphoreType.DMA,
        ],
    )
    return fn(so_flat, lse_flat, qpos_flat)


def _combine_body(lg_ref, o_ref, out_ref):
    l = lg_ref[...]
    m = jnp.max(l, axis=0, keepdims=True)
    p = jnp.exp(l - m)
    w = p / jnp.sum(p, axis=0, keepdims=True)
    out_ref[...] = jnp.sum(o_ref[...] * w[:, :, None], axis=0)


def _combine(o_un, logits):
    h, rows, d = o_un.shape
    R = 512
    out = pl.pallas_call(
        _combine_body,
        grid=(rows // R,),
        in_specs=[
            pl.BlockSpec((h, R), lambda i: (0, i)),
            pl.BlockSpec((h, R, d), lambda i: (0, i, 0)),
        ],
        out_specs=pl.BlockSpec((R, d), lambda i: (i, 0)),
        out_shape=jax.ShapeDtypeStruct((rows, d), jnp.float32),
    )(logits, o_un)
    return out


def _hash_positions(queries, keys, alpha, beta):
    q_norms = jnp.linalg.norm(queries, axis=-1, keepdims=True)
    k_norms = jnp.linalg.norm(keys, axis=-1, keepdims=True)
    MX = jnp.max(q_norms, axis=1, keepdims=True)
    MY = jnp.max(k_norms, axis=1, keepdims=True)
    M = MX + MY
    q_ext = jnp.sqrt(jnp.maximum(M ** 2 - q_norms ** 2, 0.0))
    k_ext = jnp.sqrt(jnp.maximum(M ** 2 - k_norms ** 2, 0.0))
    Q = jnp.concatenate([queries, q_ext, jnp.zeros_like(q_ext)], axis=-1)
    K = jnp.concatenate([keys, jnp.zeros_like(k_ext), k_ext], axis=-1)
    q_hashed = jnp.transpose(Q @ alpha + beta, (2, 0, 1))
    k_hashed = jnp.transpose(K @ alpha + beta, (2, 0, 1))
    q_positions = jnp.argsort(q_hashed, axis=-1).astype(jnp.int32)
    k_positions = jnp.argsort(k_hashed, axis=-1).astype(jnp.int32)
    return q_positions, k_positions


def kernel(queries, keys, values, alpha, beta):
    bs, n, d = queries.shape

    q_positions, k_positions = _hash_positions(queries, keys, alpha, beta)

    boff = (jnp.arange(bs, dtype=jnp.int32) * n)[None, :, None]
    qidx = (q_positions + boff).reshape(-1)
    kidx = (k_positions + boff).reshape(-1)

    sq, sk, sv = _gather_sorted(
        queries.reshape(bs * n, d), keys.reshape(bs * n, d),
        values.reshape(bs * n, d), qidx, kidx)

    so, lse = _cluster_attention(
        sq.reshape(-1, CLUSTER, d), sk.reshape(-1, CLUSTER, d),
        sv.reshape(-1, CLUSTER, d))

    o_un, logits = _unsort(
        so.reshape(-1, d), lse.reshape(-1), q_positions.reshape(-1), n)

    out = _combine(
        o_un.reshape(NUM_HASHES, bs * n, d),
        logits.reshape(NUM_HASHES, bs * n))
    return out.reshape(bs, n, d)


# speedup vs baseline: 7.2215x; 7.2215x over previous
import functools

import jax
import jax.numpy as jnp
from jax import lax
from jax.experimental import pallas as pl
from jax.experimental.pallas import tpu as pltpu
from jax.experimental.pallas import tpu_sc as plsc

NUM_HASHES = 4
CLUSTER = 128

SC_CORES = 2
SC_SUBCORES = 16
NW = SC_CORES * SC_SUBCORES

CH = 128


def _sc_mesh():
    return plsc.VectorSubcoreMesh(
        core_axis_name="c", subcore_axis_name="s",
        num_cores=SC_CORES, num_subcores=SC_SUBCORES)


def _wid():
    return lax.axis_index("s") * SC_CORES + lax.axis_index("c")


def _gather_sorted(qf, kf, vf, qidx, kidx):
    total, d = qf.shape[0] * NUM_HASHES, qf.shape[1]
    rows_per_w = total // NW

    def body(qf_h, kf_h, vf_h, qidx_h, kidx_h, sq_h, sk_h, sv_h,
             iq_v, ik_v, qr_v, kr_v, vr_v, sem):
        base0 = _wid() * rows_per_w

        def chunk(c, carry):
            base = base0 + c * CH
            pltpu.sync_copy(qidx_h.at[pl.ds(base, CH)], iq_v)
            pltpu.async_copy(qf_h.at[iq_v], qr_v, sem).wait()
            pltpu.sync_copy(qr_v, sq_h.at[pl.ds(base, CH)])
            pltpu.sync_copy(kidx_h.at[pl.ds(base, CH)], ik_v)
            pltpu.async_copy(kf_h.at[ik_v], kr_v, sem).wait()
            pltpu.sync_copy(kr_v, sk_h.at[pl.ds(base, CH)])
            pltpu.async_copy(vf_h.at[ik_v], vr_v, sem).wait()
            pltpu.sync_copy(vr_v, sv_h.at[pl.ds(base, CH)])
            return carry

        lax.fori_loop(0, rows_per_w // CH, chunk, 0)

    out_t = jax.ShapeDtypeStruct((total, d), jnp.float32)
    fn = pl.kernel(
        body,
        out_type=(out_t, out_t, out_t),
        mesh=_sc_mesh(),
        compiler_params=pltpu.CompilerParams(use_tc_tiling_on_sc=False),
        scratch_types=[
            pltpu.VMEM((CH,), jnp.int32),
            pltpu.VMEM((CH,), jnp.int32),
            pltpu.VMEM((CH, d), jnp.float32),
            pltpu.VMEM((CH, d), jnp.float32),
            pltpu.VMEM((CH, d), jnp.float32),
            pltpu.SemaphoreType.DMA,
        ],
    )
    return fn(qf, kf, vf, qidx, kidx)


def _attn_body(nc, sq_ref, sk_ref, sv_ref, so_ref, lse_ref):
    for j in range(nc):
        q = sq_ref[j]
        k = sk_ref[j]
        v = sv_ref[j]
        s = lax.dot_general(q, k, (((1,), (1,)), ((), ())),
                            preferred_element_type=jnp.float32)
        m = jnp.max(s, axis=1, keepdims=True)
        p = jnp.exp(s - m)
        den = jnp.sum(p, axis=1, keepdims=True)
        o = jnp.dot(p, v, preferred_element_type=jnp.float32) / den
        so_ref[j] = o
        lse_ref[j, :] = (m + jnp.log(den))[:, 0]


def _cluster_attention(sq, sk, sv):
    nclusters, c, d = sq.shape
    CB = 8
    grid = (nclusters // CB,)
    blk = pl.BlockSpec((CB, c, d), lambda i: (i, 0, 0))
    lse_blk = pl.BlockSpec((CB, c), lambda i: (i, 0))
    so, lse = pl.pallas_call(
        functools.partial(_attn_body, CB),
        grid=grid,
        in_specs=[blk, blk, blk],
        out_specs=[blk, lse_blk],
        out_shape=[
            jax.ShapeDtypeStruct((nclusters, c, d), jnp.float32),
            jax.ShapeDtypeStruct((nclusters, c), jnp.float32),
        ],
    )(sq, sk, sv)
    return so, lse


def _unsort(so_flat, lse_flat, qpos_flat, n):
    total, d = so_flat.shape
    segs = total // n
    segs_per_w = segs // NW

    def body(so_h, lse_h, qpos_h, oun_h, logits_h,
             qpos_v, lse_v, inv_v, gidx_v, log_v, rows_v, sem):
        w = _wid()

        def seg(t, carry):
            s = w * segs_per_w + t
            seg_base = s * n
            pltpu.sync_copy(qpos_h.at[pl.ds(seg_base, n)], qpos_v)
            pltpu.sync_copy(lse_h.at[pl.ds(seg_base, n)], lse_v)

            def inv_chunk(i, c2):
                idx = qpos_v[pl.ds(i * 16, 16)]
                vals = lax.iota(jnp.int32, 16) + i * 16
                plsc.store_scatter(inv_v, [idx], vals)
                return c2

            lax.fori_loop(0, n // 16, inv_chunk, 0)

            def g_chunk(i, c2):
                g = inv_v[pl.ds(i * 16, 16)]
                log_v[pl.ds(i * 16, 16)] = plsc.load_gather(lse_v, [g])
                gidx_v[pl.ds(i * 16, 16)] = g + seg_base
                return c2

            lax.fori_loop(0, n // 16, g_chunk, 0)
            pltpu.sync_copy(log_v, logits_h.at[pl.ds(seg_base, n)])

            def row_chunk(c, c2):
                base = seg_base + c * CH
                pltpu.async_copy(
                    so_h.at[gidx_v.at[pl.ds(c * CH, CH)]], rows_v, sem
                ).wait()
                pltpu.sync_copy(rows_v, oun_h.at[pl.ds(base, CH)])
                return c2

            lax.fori_loop(0, n // CH, row_chunk, 0)
            return carry

        lax.fori_loop(0, segs_per_w, seg, 0)

    fn = pl.kernel(
        body,
        out_type=(
            jax.ShapeDtypeStruct((total, d), jnp.float32),
            jax.ShapeDtypeStruct((total,), jnp.float32),
        ),
        mesh=_sc_mesh(),
        compiler_params=pltpu.CompilerParams(
            use_tc_tiling_on_sc=False, needs_layout_passes=False),
        scratch_types=[
            pltpu.VMEM((n,), jnp.int32),
            pltpu.VMEM((n,), jnp.float32),
            pltpu.VMEM((n,), jnp.int32),
            pltpu.VMEM((n,), jnp.int32),
            pltpu.VMEM((n,), jnp.float32),
            pltpu.VMEM((CH, d), jnp.float32),
            pltpu.SemaphoreType.DMA,
        ],
    )
    return fn(so_flat, lse_flat, qpos_flat)


def _combine_body(lg_ref, o_ref, out_ref):
    l = lg_ref[...]
    m = jnp.max(l, axis=0, keepdims=True)
    p = jnp.exp(l - m)
    w = p / jnp.sum(p, axis=0, keepdims=True)
    out_ref[...] = jnp.sum(o_ref[...] * w[:, :, None], axis=0)


def _combine(o_un, logits):
    h, rows, d = o_un.shape
    R = 512
    out = pl.pallas_call(
        _combine_body,
        grid=(rows // R,),
        in_specs=[
            pl.BlockSpec((h, R), lambda i: (0, i)),
            pl.BlockSpec((h, R, d), lambda i: (0, i, 0)),
        ],
        out_specs=pl.BlockSpec((R, d), lambda i: (i, 0)),
        out_shape=jax.ShapeDtypeStruct((rows, d), jnp.float32),
    )(logits, o_un)
    return out


def _hash_positions(queries, keys, alpha, beta):
    q_norms = jnp.linalg.norm(queries, axis=-1, keepdims=True)
    k_norms = jnp.linalg.norm(keys, axis=-1, keepdims=True)
    MX = jnp.max(q_norms, axis=1, keepdims=True)
    MY = jnp.max(k_norms, axis=1, keepdims=True)
    M = MX + MY
    q_ext = jnp.sqrt(jnp.maximum(M ** 2 - q_norms ** 2, 0.0))
    k_ext = jnp.sqrt(jnp.maximum(M ** 2 - k_norms ** 2, 0.0))
    Q = jnp.concatenate([queries, q_ext, jnp.zeros_like(q_ext)], axis=-1)
    K = jnp.concatenate([keys, jnp.zeros_like(k_ext), k_ext], axis=-1)
    q_hashed = jnp.transpose(Q @ alpha + beta, (2, 0, 1))
    k_hashed = jnp.transpose(K @ alpha + beta, (2, 0, 1))
    q_positions = jnp.argsort(q_hashed, axis=-1).astype(jnp.int32)
    k_positions = jnp.argsort(k_hashed, axis=-1).astype(jnp.int32)
    return q_positions, k_positions


def kernel(queries, keys, values, alpha, beta):
    bs, n, d = queries.shape

    q_positions, k_positions = _hash_positions(queries, keys, alpha, beta)

    boff = (jnp.arange(bs, dtype=jnp.int32) * n)[None, :, None]
    qidx = (q_positions + boff).reshape(-1)
    kidx = (k_positions + boff).reshape(-1)

    sq, sk, sv = _gather_sorted(
        queries.reshape(bs * n, d), keys.reshape(bs * n, d),
        values.reshape(bs * n, d), qidx, kidx)

    so, lse = _cluster_attention(
        sq.reshape(-1, CLUSTER, d), sk.reshape(-1, CLUSTER, d),
        sv.reshape(-1, CLUSTER, d))

    o_un, logits = _unsort(
        so.reshape(-1, d), lse.reshape(-1), q_positions.reshape(-1), n)

    out = _combine(
        o_un.reshape(NUM_HASHES, bs * n, d),
        logits.reshape(NUM_HASHES, bs * n))
    return out.reshape(bs, n, d)


# speedup vs baseline: 8.4834x; 1.1747x over previous
import functools

import jax
import jax.numpy as jnp
from jax import lax
from jax.experimental import pallas as pl
from jax.experimental.pallas import tpu as pltpu
from jax.experimental.pallas import tpu_sc as plsc

NUM_HASHES = 4
CLUSTER = 128

SC_CORES = 2
SC_SUBCORES = 16
NW = SC_CORES * SC_SUBCORES

CH = 128


def _sc_mesh():
    return plsc.VectorSubcoreMesh(
        core_axis_name="c", subcore_axis_name="s",
        num_cores=SC_CORES, num_subcores=SC_SUBCORES)


def _wid():
    return lax.axis_index("s") * SC_CORES + lax.axis_index("c")


def _gather_sorted(qf, kf, vf, qidx, kidx):
    total, d = qf.shape[0] * NUM_HASHES, qf.shape[1]
    rows_per_w = total // NW

    def body(qf_h, kf_h, vf_h, qidx_h, kidx_h, sq_h, sk_h, sv_h,
             iq_v, ik_v, qr_v, kr_v, vr_v, sem):
        base0 = _wid() * rows_per_w

        def chunk(c, carry):
            base = base0 + c * CH
            pltpu.sync_copy(qidx_h.at[pl.ds(base, CH)], iq_v)
            pltpu.async_copy(qf_h.at[iq_v], qr_v, sem).wait()
            pltpu.sync_copy(qr_v, sq_h.at[pl.ds(base, CH)])
            pltpu.sync_copy(kidx_h.at[pl.ds(base, CH)], ik_v)
            pltpu.async_copy(kf_h.at[ik_v], kr_v, sem).wait()
            pltpu.sync_copy(kr_v, sk_h.at[pl.ds(base, CH)])
            pltpu.async_copy(vf_h.at[ik_v], vr_v, sem).wait()
            pltpu.sync_copy(vr_v, sv_h.at[pl.ds(base, CH)])
            return carry

        lax.fori_loop(0, rows_per_w // CH, chunk, 0)

    out_t = jax.ShapeDtypeStruct((total, d), jnp.float32)
    fn = pl.kernel(
        body,
        out_type=(out_t, out_t, out_t),
        mesh=_sc_mesh(),
        compiler_params=pltpu.CompilerParams(use_tc_tiling_on_sc=False),
        scratch_types=[
            pltpu.VMEM((CH,), jnp.int32),
            pltpu.VMEM((CH,), jnp.int32),
            pltpu.VMEM((CH, d), jnp.float32),
            pltpu.VMEM((CH, d), jnp.float32),
            pltpu.VMEM((CH, d), jnp.float32),
            pltpu.SemaphoreType.DMA,
        ],
    )
    return fn(qf, kf, vf, qidx, kidx)


def _attn_body(sq_ref, sk_ref, sv_ref, so_ref, lse_ref):
    qs = sq_ref[...]
    ks = sk_ref[...]
    vs = sv_ref[...]
    s = lax.dot_general(qs, ks, (((2,), (2,)), ((0,), (0,))),
                        preferred_element_type=jnp.float32)
    m = jnp.max(s, axis=2, keepdims=True)
    p = jnp.exp(s - m)
    den = jnp.sum(p, axis=2, keepdims=True)
    o = lax.dot_general(p, vs, (((2,), (1,)), ((0,), (0,))),
                        preferred_element_type=jnp.float32)
    so_ref[...] = o / den
    lse_ref[...] = (m + jnp.log(den))[:, :, 0]


def _cluster_attention(sq, sk, sv):
    nclusters, c, d = sq.shape
    CB = 16
    grid = (nclusters // CB,)
    blk = pl.BlockSpec((CB, c, d), lambda i: (i, 0, 0))
    lse_blk = pl.BlockSpec((CB, c), lambda i: (i, 0))
    so, lse = pl.pallas_call(
        _attn_body,
        grid=grid,
        in_specs=[blk, blk, blk],
        out_specs=[blk, lse_blk],
        out_shape=[
            jax.ShapeDtypeStruct((nclusters, c, d), jnp.float32),
            jax.ShapeDtypeStruct((nclusters, c), jnp.float32),
        ],
    )(sq, sk, sv)
    return so, lse


def _unsort(so_flat, lse_flat, qpos_flat, n):
    total, d = so_flat.shape
    segs = total // n
    segs_per_w = segs // NW

    def body(so_h, lse_h, qpos_h, oun_h, logits_h,
             qpos_v, lse_v, inv_v, gidx_v, log_v, rows_v, sem):
        w = _wid()

        def seg(t, carry):
            s = w * segs_per_w + t
            seg_base = s * n
            pltpu.sync_copy(qpos_h.at[pl.ds(seg_base, n)], qpos_v)
            pltpu.sync_copy(lse_h.at[pl.ds(seg_base, n)], lse_v)

            def inv_chunk(i, c2):
                idx = qpos_v[pl.ds(i * 16, 16)]
                vals = lax.iota(jnp.int32, 16) + i * 16
                plsc.store_scatter(inv_v, [idx], vals)
                return c2

            lax.fori_loop(0, n // 16, inv_chunk, 0)

            def g_chunk(i, c2):
                g = inv_v[pl.ds(i * 16, 16)]
                log_v[pl.ds(i * 16, 16)] = plsc.load_gather(lse_v, [g])
                gidx_v[pl.ds(i * 16, 16)] = g + seg_base
                return c2

            lax.fori_loop(0, n // 16, g_chunk, 0)
            pltpu.sync_copy(log_v, logits_h.at[pl.ds(seg_base, n)])

            def row_chunk(c, c2):
                base = seg_base + c * CH
                pltpu.async_copy(
                    so_h.at[gidx_v.at[pl.ds(c * CH, CH)]], rows_v, sem
                ).wait()
                pltpu.sync_copy(rows_v, oun_h.at[pl.ds(base, CH)])
                return c2

            lax.fori_loop(0, n // CH, row_chunk, 0)
            return carry

        lax.fori_loop(0, segs_per_w, seg, 0)

    fn = pl.kernel(
        body,
        out_type=(
            jax.ShapeDtypeStruct((total, d), jnp.float32),
            jax.ShapeDtypeStruct((total,), jnp.float32),
        ),
        mesh=_sc_mesh(),
        compiler_params=pltpu.CompilerParams(
            use_tc_tiling_on_sc=False, needs_layout_passes=False),
        scratch_types=[
            pltpu.VMEM((n,), jnp.int32),
            pltpu.VMEM((n,), jnp.float32),
            pltpu.VMEM((n,), jnp.int32),
            pltpu.VMEM((n,), jnp.int32),
            pltpu.VMEM((n,), jnp.float32),
            pltpu.VMEM((CH, d), jnp.float32),
            pltpu.SemaphoreType.DMA,
        ],
    )
    return fn(so_flat, lse_flat, qpos_flat)


def _combine_body(lg_ref, o_ref, out_ref):
    l = lg_ref[...]
    m = jnp.max(l, axis=0, keepdims=True)
    p = jnp.exp(l - m)
    w = p / jnp.sum(p, axis=0, keepdims=True)
    out_ref[...] = jnp.sum(o_ref[...] * w[:, :, None], axis=0)


def _combine(o_un, logits):
    h, rows, d = o_un.shape
    R = 512
    out = pl.pallas_call(
        _combine_body,
        grid=(rows // R,),
        in_specs=[
            pl.BlockSpec((h, R), lambda i: (0, i)),
            pl.BlockSpec((h, R, d), lambda i: (0, i, 0)),
        ],
        out_specs=pl.BlockSpec((R, d), lambda i: (i, 0)),
        out_shape=jax.ShapeDtypeStruct((rows, d), jnp.float32),
    )(logits, o_un)
    return out


def _hash_positions(queries, keys, alpha, beta):
    q_norms = jnp.linalg.norm(queries, axis=-1, keepdims=True)
    k_norms = jnp.linalg.norm(keys, axis=-1, keepdims=True)
    MX = jnp.max(q_norms, axis=1, keepdims=True)
    MY = jnp.max(k_norms, axis=1, keepdims=True)
    M = MX + MY
    q_ext = jnp.sqrt(jnp.maximum(M ** 2 - q_norms ** 2, 0.0))
    k_ext = jnp.sqrt(jnp.maximum(M ** 2 - k_norms ** 2, 0.0))
    Q = jnp.concatenate([queries, q_ext, jnp.zeros_like(q_ext)], axis=-1)
    K = jnp.concatenate([keys, jnp.zeros_like(k_ext), k_ext], axis=-1)
    q_hashed = jnp.transpose(Q @ alpha + beta, (2, 0, 1))
    k_hashed = jnp.transpose(K @ alpha + beta, (2, 0, 1))
    q_positions = jnp.argsort(q_hashed, axis=-1).astype(jnp.int32)
    k_positions = jnp.argsort(k_hashed, axis=-1).astype(jnp.int32)
    return q_positions, k_positions


def kernel(queries, keys, values, alpha, beta):
    bs, n, d = queries.shape

    q_positions, k_positions = _hash_positions(queries, keys, alpha, beta)

    boff = (jnp.arange(bs, dtype=jnp.int32) * n)[None, :, None]
    qidx = (q_positions + boff).reshape(-1)
    kidx = (k_positions + boff).reshape(-1)

    sq, sk, sv = _gather_sorted(
        queries.reshape(bs * n, d), keys.reshape(bs * n, d),
        values.reshape(bs * n, d), qidx, kidx)

    so, lse = _cluster_attention(
        sq.reshape(-1, CLUSTER, d), sk.reshape(-1, CLUSTER, d),
        sv.reshape(-1, CLUSTER, d))

    o_un, logits = _unsort(
        so.reshape(-1, d), lse.reshape(-1), q_positions.reshape(-1), n)

    out = _combine(
        o_un.reshape(NUM_HASHES, bs * n, d),
        logits.reshape(NUM_HASHES, bs * n))
    return out.reshape(bs, n, d)
